# baseline (speedup 1.0000x reference)
import functools

import jax
import jax.numpy as jnp
from jax import lax
from jax.experimental import pallas as pl
from jax.experimental.pallas import tpu as pltpu
from jax.experimental.pallas import tpu_sc as plsc

_V = 1000000
_D = 64
_H = 256
_OUT = 128
_B = 16384
_N = 2 * _B

_TBLK = 10240
_M = 256000
_NLINEBLK = _M // _TBLK
_QB = _NLINEBLK
_LASTBLK = (_V + _TBLK - 1) // _TBLK - 1

_NC = 2
_NS = 16
_NW = _NC * _NS
_PER_W = _N // _NW
_CHUNK = 256
_NCHUNK = _PER_W // _CHUNK

_PARALLEL = pltpu.CompilerParams(dimension_semantics=("parallel",))


def _rne_bf16_bits(v):
    u = lax.bitcast_convert_type(v, jnp.uint32)
    return u + jnp.uint32(0x7FFF) + ((u >> jnp.uint32(16)) & jnp.uint32(1))


def _transpose_body(a_ref, b_ref, c_ref, d_ref, dst_ref):
    stacked = jnp.concatenate(
        [a_ref[...], b_ref[...], c_ref[...], d_ref[...]], axis=0)
    t = jnp.transpose(stacked, (1, 0))
    lo = _rne_bf16_bits(t[:, 0:2 * _D])
    hi = _rne_bf16_bits(t[:, 2 * _D:4 * _D])
    word = ((lo >> jnp.uint32(16)) & jnp.uint32(0xFFFF)) | (
        hi & jnp.uint32(0xFFFF0000))
    dst_ref[...] = lax.bitcast_convert_type(word, jnp.float32)


def _build_packed(emb_t):
    return pl.pallas_call(
        _transpose_body,
        grid=(_NLINEBLK,),
        in_specs=[
            pl.BlockSpec((_D, _TBLK), lambda i: (0, i)),
            pl.BlockSpec((_D, _TBLK), lambda i: (0, i + _QB)),
            pl.BlockSpec((_D, _TBLK), lambda i: (0, i + 2 * _QB)),
            pl.BlockSpec(
                (_D, _TBLK),
                lambda i: (0, jnp.minimum(i + 3 * _QB, _LASTBLK))),
        ],
        out_specs=pl.BlockSpec((_TBLK, 2 * _D), lambda i: (i, 0)),
        out_shape=jax.ShapeDtypeStruct((_M, 2 * _D), jnp.float32),
        compiler_params=_PARALLEL,
    )(emb_t, emb_t, emb_t, emb_t)


def _sc_gather(table, idx):
    mesh = plsc.VectorSubcoreMesh(core_axis_name="c", subcore_axis_name="s")

    @functools.partial(
        pl.kernel,
        mesh=mesh,
        out_type=jax.ShapeDtypeStruct((_N, 2 * _D), jnp.float32),
        scratch_types=[
            pltpu.VMEM((_PER_W,), jnp.int32),
            pltpu.VMEM((_CHUNK, 2 * _D), jnp.float32),
            pltpu.VMEM((_CHUNK, 2 * _D), jnp.float32),
            pltpu.SemaphoreType.DMA,
            pltpu.SemaphoreType.DMA,
        ],
    )
    def k(tab_hbm, idx_hbm, out_hbm, idx_v, buf0, buf1, sem0, sem1):
        wid = lax.axis_index("s") * _NC + lax.axis_index("c")
        base = wid * _PER_W
        pltpu.sync_copy(idx_hbm.at[pl.ds(base, _PER_W)], idx_v)
        bufs = (buf0, buf1)
        sems = (sem0, sem1)
        cps = []
        for c in range(_NCHUNK):
            b = c % 2
            cp = pltpu.make_async_copy(
                tab_hbm.at[idx_v.at[pl.ds(c * _CHUNK, _CHUNK)]], bufs[b], sems[b])
            cp.start()
            cps.append(cp)
            if c >= 1:
                cps[c - 1].wait()
                pltpu.sync_copy(
                    bufs[(c - 1) % 2],
                    out_hbm.at[pl.ds(base + (c - 1) * _CHUNK, _CHUNK)])
        cps[_NCHUNK - 1].wait()
        pltpu.sync_copy(
            bufs[(_NCHUNK - 1) % 2],
            out_hbm.at[pl.ds(base + (_NCHUNK - 1) * _CHUNK, _CHUNK)])

    return k(table, idx)


_BLK = 2048


def _unpack_select(g_ref, q_col):
    u = lax.bitcast_convert_type(g_ref[...], jnp.uint32)
    lo = lax.bitcast_convert_type(u << jnp.uint32(16), jnp.float32)
    hi = lax.bitcast_convert_type(u & jnp.uint32(0xFFFF0000), jnp.float32)
    e01 = jnp.where(q_col == 1, lo[:, _D:2 * _D], lo[:, 0:_D])
    e23 = jnp.where(q_col == 3, hi[:, _D:2 * _D], hi[:, 0:_D])
    return jnp.where(q_col >= 2, e23, e01)


def _mlp_body(g1_ref, g2_ref, q_ref, w1_ref, b1_ref, w2_ref, b2_ref, o_ref):
    e1 = _unpack_select(g1_ref, q_ref[:, 0:1])
    e2 = _unpack_select(g2_ref, q_ref[:, 1:2])
    cat = jnp.concatenate([e1, e2], axis=1)
    h = lax.dot_general(
        cat, w1_ref[...], (((1,), (1,)), ((), ())),
        preferred_element_type=jnp.float32,
    )
    h = jnp.maximum(h + b1_ref[...], 0.0)
    o_ref[...] = lax.dot_general(
        h, w2_ref[...], (((1,), (1,)), ((), ())),
        preferred_element_type=jnp.float32,
    ) + b2_ref[...]


def _mlp(rows, qsel, W1, b1, W2, b2):
    nblk = _B // _BLK
    return pl.pallas_call(
        _mlp_body,
        grid=(nblk,),
        in_specs=[
            pl.BlockSpec((_BLK, 2 * _D), lambda i: (i, 0)),
            pl.BlockSpec((_BLK, 2 * _D), lambda i: (i + nblk, 0)),
            pl.BlockSpec((_BLK, 2), lambda i: (i, 0)),
            pl.BlockSpec((_H, 2 * _D), lambda i: (0, 0)),
            pl.BlockSpec((1, _H), lambda i: (0, 0)),
            pl.BlockSpec((_OUT, _H), lambda i: (0, 0)),
            pl.BlockSpec((1, _OUT), lambda i: (0, 0)),
        ],
        out_specs=pl.BlockSpec((_BLK, _OUT), lambda i: (i, 0)),
        out_shape=jax.ShapeDtypeStruct((_B, _OUT), jnp.float32),
        compiler_params=_PARALLEL,
    )(rows, rows, qsel, W1, b1.reshape(1, _H), W2, b2.reshape(1, _OUT))


def kernel(x, emb, W1, b1, W2, b2):
    xi = x.astype(jnp.int32)
    idx = xi.T.reshape(_N)
    q = idx // _M
    line_idx = idx - q * _M
    qsel = (xi // _M).astype(jnp.int32)
    emb_t = emb.T
    table = _build_packed(emb_t)
    rows = _sc_gather(table, line_idx)
    return _mlp(rows, qsel, W1, b1, W2, b2)


# speedup vs baseline: 1.1026x; 1.1026x over previous
import functools

import jax
import jax.numpy as jnp
from jax import lax
from jax.experimental import pallas as pl
from jax.experimental.pallas import tpu as pltpu
from jax.experimental.pallas import tpu_sc as plsc

_V = 1000000
_D = 64
_H = 256
_OUT = 128
_B = 16384
_N = 2 * _B

_TBLK = 8192
_M = 262144
_NLINEBLK = _M // _TBLK
_QB = _NLINEBLK
_LASTBLK = (_V + _TBLK - 1) // _TBLK - 1

_NC = 2
_NS = 16
_NW = _NC * _NS
_PER_W = _N // _NW
_CHUNK = 256
_NCHUNK = _PER_W // _CHUNK

_PARALLEL = pltpu.CompilerParams(dimension_semantics=("parallel",))


def _rne_bf16_bits(v):
    u = lax.bitcast_convert_type(v, jnp.uint32)
    return u + jnp.uint32(0x7FFF) + ((u >> jnp.uint32(16)) & jnp.uint32(1))


def _transpose_body(a_ref, b_ref, c_ref, d_ref, dst_ref):
    stacked = jnp.concatenate(
        [a_ref[...], b_ref[...], c_ref[...], d_ref[...]], axis=0)
    t = jnp.transpose(stacked, (1, 0))
    lo = _rne_bf16_bits(t[:, 0:2 * _D])
    hi = _rne_bf16_bits(t[:, 2 * _D:4 * _D])
    word = ((lo >> jnp.uint32(16)) & jnp.uint32(0xFFFF)) | (
        hi & jnp.uint32(0xFFFF0000))
    dst_ref[...] = lax.bitcast_convert_type(word, jnp.float32)


def _build_packed(emb_t):
    return pl.pallas_call(
        _transpose_body,
        grid=(_NLINEBLK,),
        in_specs=[
            pl.BlockSpec((_D, _TBLK), lambda i: (0, i)),
            pl.BlockSpec((_D, _TBLK), lambda i: (0, i + _QB)),
            pl.BlockSpec((_D, _TBLK), lambda i: (0, i + 2 * _QB)),
            pl.BlockSpec(
                (_D, _TBLK),
                lambda i: (0, jnp.minimum(i + 3 * _QB, _LASTBLK))),
        ],
        out_specs=pl.BlockSpec((_TBLK, 2 * _D), lambda i: (i, 0)),
        out_shape=jax.ShapeDtypeStruct((_M, 2 * _D), jnp.float32),
        compiler_params=_PARALLEL,
    )(emb_t, emb_t, emb_t, emb_t)


def _sc_gather(table, idx):
    mesh = plsc.VectorSubcoreMesh(core_axis_name="c", subcore_axis_name="s")

    @functools.partial(
        pl.kernel,
        mesh=mesh,
        out_type=jax.ShapeDtypeStruct((_N, 2 * _D), jnp.float32),
        scratch_types=[
            pltpu.VMEM((_PER_W,), jnp.int32),
            pltpu.VMEM((_CHUNK, 2 * _D), jnp.float32),
            pltpu.VMEM((_CHUNK, 2 * _D), jnp.float32),
            pltpu.SemaphoreType.DMA,
            pltpu.SemaphoreType.DMA,
        ],
    )
    def k(tab_hbm, idx_hbm, out_hbm, idx_v, buf0, buf1, sem0, sem1):
        wid = lax.axis_index("s") * _NC + lax.axis_index("c")
        base = wid * _PER_W
        pltpu.sync_copy(idx_hbm.at[pl.ds(base, _PER_W)], idx_v)
        bufs = (buf0, buf1)
        sems = (sem0, sem1)
        cps = []
        for c in range(_NCHUNK):
            b = c % 2
            cp = pltpu.make_async_copy(
                tab_hbm.at[idx_v.at[pl.ds(c * _CHUNK, _CHUNK)]], bufs[b], sems[b])
            cp.start()
            cps.append(cp)
            if c >= 1:
                cps[c - 1].wait()
                pltpu.sync_copy(
                    bufs[(c - 1) % 2],
                    out_hbm.at[pl.ds(base + (c - 1) * _CHUNK, _CHUNK)])
        cps[_NCHUNK - 1].wait()
        pltpu.sync_copy(
            bufs[(_NCHUNK - 1) % 2],
            out_hbm.at[pl.ds(base + (_NCHUNK - 1) * _CHUNK, _CHUNK)])

    return k(table, idx)


_BLK = 2048


def _unpack_select(g_ref, q_col):
    u = lax.bitcast_convert_type(g_ref[...], jnp.uint32)
    lo = lax.bitcast_convert_type(u << jnp.uint32(16), jnp.float32)
    hi = lax.bitcast_convert_type(u & jnp.uint32(0xFFFF0000), jnp.float32)
    e01 = jnp.where(q_col == 1, lo[:, _D:2 * _D], lo[:, 0:_D])
    e23 = jnp.where(q_col == 3, hi[:, _D:2 * _D], hi[:, 0:_D])
    return jnp.where(q_col >= 2, e23, e01)


def _mlp_body(g1_ref, g2_ref, q_ref, w1_ref, b1_ref, w2_ref, b2_ref, o_ref):
    e1 = _unpack_select(g1_ref, q_ref[:, 0:1])
    e2 = _unpack_select(g2_ref, q_ref[:, 1:2])
    cat = jnp.concatenate([e1, e2], axis=1)
    h = lax.dot_general(
        cat, w1_ref[...], (((1,), (1,)), ((), ())),
        preferred_element_type=jnp.float32,
    )
    h = jnp.maximum(h + b1_ref[...], 0.0)
    o_ref[...] = lax.dot_general(
        h, w2_ref[...], (((1,), (1,)), ((), ())),
        preferred_element_type=jnp.float32,
    ) + b2_ref[...]


def _mlp(rows, qsel, W1, b1, W2, b2):
    nblk = _B // _BLK
    return pl.pallas_call(
        _mlp_body,
        grid=(nblk,),
        in_specs=[
            pl.BlockSpec((_BLK, 2 * _D), lambda i: (i, 0)),
            pl.BlockSpec((_BLK, 2 * _D), lambda i: (i + nblk, 0)),
            pl.BlockSpec((_BLK, 2), lambda i: (i, 0)),
            pl.BlockSpec((_H, 2 * _D), lambda i: (0, 0)),
            pl.BlockSpec((1, _H), lambda i: (0, 0)),
            pl.BlockSpec((_OUT, _H), lambda i: (0, 0)),
            pl.BlockSpec((1, _OUT), lambda i: (0, 0)),
        ],
        out_specs=pl.BlockSpec((_BLK, _OUT), lambda i: (i, 0)),
        out_shape=jax.ShapeDtypeStruct((_B, _OUT), jnp.float32),
        compiler_params=_PARALLEL,
    )(rows, rows, qsel, W1, b1.reshape(1, _H), W2, b2.reshape(1, _OUT))


def kernel(x, emb, W1, b1, W2, b2):
    xi = x.astype(jnp.int32)
    idx = xi.T.reshape(_N)
    line_idx = idx & (_M - 1)
    qsel = xi >> 18
    emb_t = emb.T
    table = _build_packed(emb_t)
    rows = _sc_gather(table, line_idx)
    return _mlp(rows, qsel, W1, b1, W2, b2)
